# Initial kernel scaffold; baseline (speedup 1.0000x reference)
import jax
import jax.numpy as jnp
from jax.experimental import pallas as pl


def kernel(data, net_index, nets, W_gnn, b_gnn, W_lin, b_lin):
    raise NotImplementedError("write your pallas kernel here")


# speedup vs baseline: 183.5761x; 183.5761x over previous
import jax
import jax.numpy as jnp
from jax.experimental import pallas as pl
from jax.experimental.pallas import tpu as pltpu

_TAU = 0.5


def _body(idx_ref, nets_ref, e_ref, x_ref, wg_ref, bg_ref, wl_ref, bl_ref,
          out_ref, emb_ref, ret_ref, ys_ref):
    n, d = x_ref.shape[1], x_ref.shape[2]
    nfeat = wg_ref.shape[1]
    ncls = wl_ref.shape[1]
    logits = nets_ref[0]
    g = -jnp.log(e_ref[0])
    s = jax.nn.sigmoid((logits + g) * (1.0 / _TAU))
    ys = s * 0.5 + s.T * 0.5
    ys_ref[0] = ys
    ret_ref[0] = jnp.ones_like(ys)
    xs = jnp.sum(x_ref[0], axis=0, keepdims=True)
    emb_row = jnp.maximum(
        jnp.dot(xs, wg_ref[...], preferred_element_type=jnp.float32)
        + bg_ref[...], 0.0)
    emb_ref[0] = jnp.broadcast_to(emb_row, (n, nfeat))
    out_row = (jnp.dot(emb_row, wl_ref[...], preferred_element_type=jnp.float32)
               + bl_ref[...])
    out_ref[0] = jnp.broadcast_to(out_row, (n, ncls))


def kernel(data, net_index, nets, W_gnn, b_gnn, W_lin, b_lin):
    B, N, D = data.shape
    F = W_gnn.shape[1]
    C = W_lin.shape[1]
    e = jax.random.exponential(jax.random.key(42), (B, N, N), dtype=jnp.float32)
    grid_spec = pltpu.PrefetchScalarGridSpec(
        num_scalar_prefetch=1,
        grid=(B,),
        in_specs=[
            pl.BlockSpec((1, N, N), lambda b, idx: (idx[b], 0, 0)),
            pl.BlockSpec((1, N, N), lambda b, idx: (b, 0, 0)),
            pl.BlockSpec((1, N, D), lambda b, idx: (b, 0, 0)),
            pl.BlockSpec((D, F), lambda b, idx: (0, 0)),
            pl.BlockSpec((1, F), lambda b, idx: (0, 0)),
            pl.BlockSpec((F, C), lambda b, idx: (0, 0)),
            pl.BlockSpec((1, C), lambda b, idx: (0, 0)),
        ],
        out_specs=[
            pl.BlockSpec((1, N, C), lambda b, idx: (b, 0, 0)),
            pl.BlockSpec((1, N, F), lambda b, idx: (b, 0, 0)),
            pl.BlockSpec((1, N, N), lambda b, idx: (b, 0, 0)),
            pl.BlockSpec((1, N, N), lambda b, idx: (b, 0, 0)),
        ],
    )
    out_shapes = [
        jax.ShapeDtypeStruct((B, N, C), jnp.float32),
        jax.ShapeDtypeStruct((B, N, F), jnp.float32),
        jax.ShapeDtypeStruct((B, N, N), jnp.float32),
        jax.ShapeDtypeStruct((B, N, N), jnp.float32),
    ]
    output, embeddings, ret, y_soft = pl.pallas_call(
        _body,
        grid_spec=grid_spec,
        out_shape=out_shapes,
        compiler_params=pltpu.CompilerParams(
            dimension_semantics=("arbitrary",)),
    )(net_index, nets, e, data,
      W_gnn, b_gnn.reshape(1, F), W_lin, b_lin.reshape(1, C))
    return (output, embeddings, ret, y_soft)
